# baseline (speedup 1.0000x reference)
import jax
import jax.numpy as jnp
from jax import lax
from jax.experimental import pallas as pl
from jax.experimental.pallas import tpu as pltpu


def _make_body(actions: int, s_dim: int):
    def _body(s_ref, a_ref, w1f_ref, w2_ref, b2_ref, o_ref):
        s = s_ref[...].astype(jnp.bfloat16)
        a = a_ref[...]
        iota = lax.broadcasted_iota(
            jnp.int32, (a.shape[0], actions), 1).astype(jnp.float32)
        a_bc = jnp.dot(a, jnp.ones((1, actions), jnp.bfloat16),
                       preferred_element_type=jnp.float32)
        onehot = (a_bc == iota).astype(jnp.bfloat16)

        h = jnp.dot(s, w1f_ref[pl.ds(0, s_dim), :],
                    preferred_element_type=jnp.float32)
        h = h + jnp.dot(onehot, w1f_ref[pl.ds(s_dim, actions), :],
                        preferred_element_type=jnp.float32)
        h = jnp.maximum(h.astype(jnp.bfloat16), jnp.bfloat16(0.0))

        out = jnp.dot(h, w2_ref[...], preferred_element_type=jnp.float32)
        o_ref[...] = out + b2_ref[...]

    return _body


def kernel(s, a, w1, b1, w2, b2):
    T, S = s.shape
    H = w1.shape[1]
    O = w2.shape[1]
    A = w1.shape[0] - S

    b1 = jnp.reshape(b1, (1, H)).astype(jnp.float32)
    b2 = jnp.reshape(b2, (1, O)).astype(jnp.float32)
    w1f = jnp.concatenate([w1[:S], w1[S:] + b1], axis=0).astype(jnp.bfloat16)
    w2b = w2.astype(jnp.bfloat16)

    TM = 8192
    grid = (pl.cdiv(T, TM),)

    return pl.pallas_call(
        _make_body(A, S),
        out_shape=jax.ShapeDtypeStruct((T, O), jnp.float32),
        grid=grid,
        in_specs=[
            pl.BlockSpec((TM, S), lambda i: (i, 0)),
            pl.BlockSpec((TM, 1), lambda i: (i, 0)),
            pl.BlockSpec((S + A, H), lambda i: (0, 0)),
            pl.BlockSpec((H, O), lambda i: (0, 0)),
            pl.BlockSpec((1, O), lambda i: (0, 0)),
        ],
        out_specs=pl.BlockSpec((TM, O), lambda i: (i, 0)),
        compiler_params=pltpu.CompilerParams(
            dimension_semantics=("parallel",)),
    )(s, a.astype(jnp.bfloat16), w1f, w2b, b2)


# speedup vs baseline: 1.1399x; 1.1399x over previous
import jax
import jax.numpy as jnp
from jax import lax
from jax.experimental import pallas as pl
from jax.experimental.pallas import tpu as pltpu


def _make_body(actions: int, s_dim: int):
    def _body(s_ref, a_ref, w1f_ref, w2_ref, b2_ref, o_ref):
        s = s_ref[...].astype(jnp.bfloat16)
        a = a_ref[...]
        iota = lax.broadcasted_iota(
            jnp.int32, (a.shape[0], actions), 1).astype(jnp.bfloat16)
        onehot = (a == iota).astype(jnp.bfloat16)

        h = jnp.dot(s, w1f_ref[pl.ds(0, s_dim), :],
                    preferred_element_type=jnp.float32)
        h = h + jnp.dot(onehot, w1f_ref[pl.ds(s_dim, actions), :],
                        preferred_element_type=jnp.float32)
        h = jnp.maximum(h, 0.0).astype(jnp.bfloat16)

        out = jnp.dot(h, w2_ref[...], preferred_element_type=jnp.float32)
        o_ref[...] = out + b2_ref[...]

    return _body


def kernel(s, a, w1, b1, w2, b2):
    T, S = s.shape
    H = w1.shape[1]
    O = w2.shape[1]
    A = w1.shape[0] - S

    b1 = jnp.reshape(b1, (1, H)).astype(jnp.float32)
    b2 = jnp.reshape(b2, (1, O)).astype(jnp.float32)
    w1f = jnp.concatenate([w1[:S], w1[S:] + b1], axis=0).astype(jnp.bfloat16)
    w2b = w2.astype(jnp.bfloat16)

    TM = 8192
    grid = (pl.cdiv(T, TM),)

    return pl.pallas_call(
        _make_body(A, S),
        out_shape=jax.ShapeDtypeStruct((T, O), jnp.float32),
        grid=grid,
        in_specs=[
            pl.BlockSpec((TM, S), lambda i: (i, 0)),
            pl.BlockSpec((TM, 1), lambda i: (i, 0)),
            pl.BlockSpec((S + A, H), lambda i: (0, 0)),
            pl.BlockSpec((H, O), lambda i: (0, 0)),
            pl.BlockSpec((1, O), lambda i: (0, 0)),
        ],
        out_specs=pl.BlockSpec((TM, O), lambda i: (i, 0)),
        compiler_params=pltpu.CompilerParams(
            dimension_semantics=("parallel",)),
    )(s, a.astype(jnp.bfloat16), w1f, w2b, b2)
